# baseline (speedup 1.0000x reference)
import functools

import jax
import jax.numpy as jnp
from jax import lax
from jax.experimental import pallas as pl

_KU = (0x6D3E048F, 0x1022172D)
_KS = (0x03D7B32D, 0xADD083F4)

_EPS = 1e-12
_TINY = float(jnp.finfo(jnp.float32).tiny)
_ROT = ((13, 15, 26, 6), (17, 29, 16, 24))

_ROWS_PER_STEP = 16
_CHUNK = 1024


def _threefry_bits(cnt, k0, k1):
    ks = (k0, k1, k0 ^ k1 ^ 0x1BD11BDA)
    x0 = jnp.full(cnt.shape, jnp.uint32(k0))
    x1 = cnt + jnp.uint32(k1)
    for b in range(5):
        for r in _ROT[b % 2]:
            x0 = x0 + x1
            x1 = (x1 << jnp.uint32(r)) | (x1 >> jnp.uint32(32 - r))
            x1 = x1 ^ x0
        x0 = x0 + jnp.uint32(ks[(b + 1) % 3])
        x1 = x1 + jnp.uint32((ks[(b + 2) % 3] + b + 1) & 0xFFFFFFFF)
    return x0 ^ x1


def _bits_to_unit_float(bits):
    f = lax.bitcast_convert_type(
        (bits >> jnp.uint32(9)) | jnp.uint32(0x3F800000), jnp.float32
    )
    return f - jnp.float32(1.0)


def _gumbel_logits(x, cnt):
    f = _bits_to_unit_float(_threefry_bits(cnt, *_KU))
    u = f * jnp.float32(0.999 - _EPS) + jnp.float32(_EPS)
    return x + (-jnp.log(-jnp.log(u)))


def _gumbel2(cnt):
    f = _bits_to_unit_float(_threefry_bits(cnt, *_KS))
    u = jnp.maximum(
        f * jnp.float32(1.0 - _TINY) + jnp.float32(_TINY), jnp.float32(_TINY)
    )
    return -jnp.log(-jnp.log(u))


def _stgs_body(x_ref, y_ref, ids_ref, *, vocab):
    i = pl.program_id(0)
    R, C = _ROWS_PER_STEP, _CHUNK
    n_chunks = vocab // C
    tail_start = n_chunks * C
    tail = vocab - tail_start

    row_base = (
        lax.broadcasted_iota(jnp.int32, (R, 1), 0) + i * R
    ) * vocab

    def cnt_for(col):
        return (row_base + col).astype(jnp.uint32)

    def col_iota(j, width):
        return lax.broadcasted_iota(jnp.int32, (R, width), 1) + j * C

    def chunk_a(j, carry, width):
        m, s = carry
        gl = _gumbel_logits(
            x_ref[:, j * C:j * C + width], cnt_for(col_iota(j, width))
        )
        y_ref[:, j * C:j * C + width] = gl
        m_new = jnp.maximum(m, jnp.max(gl, axis=1, keepdims=True))
        s = s * jnp.exp(m - m_new) + jnp.sum(
            jnp.exp(gl - m_new), axis=1, keepdims=True
        )
        return m_new, s

    carry = (
        jnp.full((R, 1), -jnp.inf, jnp.float32),
        jnp.zeros((R, 1), jnp.float32),
    )
    for j in range(n_chunks):
        carry = chunk_a(j, carry, C)
    if tail:
        carry = chunk_a(n_chunks, carry, tail)
    m, s = carry
    rinv = jnp.float32(1.0) / s

    def chunk_b(j, carry, width):
        vmax, vidx = carry
        y = jnp.exp(y_ref[:, j * C:j * C + width] - m) * rinv
        y_ref[:, j * C:j * C + width] = y
        col = col_iota(j, width)
        vals = jnp.log(y + jnp.float32(_EPS)) + _gumbel2(cnt_for(col))
        lm = jnp.max(vals, axis=1, keepdims=True)
        lidx = jnp.min(
            jnp.where(vals == lm, col, jnp.int32(vocab)), axis=1, keepdims=True
        )
        upd = lm > vmax
        return jnp.where(upd, lm, vmax), jnp.where(upd, lidx, vidx)

    carry = (
        jnp.full((R, 1), -jnp.inf, jnp.float32),
        jnp.zeros((R, 1), jnp.int32),
    )
    for j in range(n_chunks):
        carry = chunk_b(j, carry, C)
    if tail:
        carry = chunk_b(n_chunks, carry, tail)

    ids_ref[...] = jnp.broadcast_to(carry[1], (R, 128))


@functools.partial(jax.jit, static_argnames=("interpret",))
def kernel(x, interpret=False):
    b0, b1, vocab = x.shape
    rows = b0 * b1
    rows_per_step = _ROWS_PER_STEP
    grid = rows // rows_per_step
    x2 = x.reshape(rows, vocab)

    y2, ids2 = pl.pallas_call(
        functools.partial(_stgs_body, vocab=vocab),
        grid=(grid,),
        in_specs=[
            pl.BlockSpec((rows_per_step, vocab), lambda i: (i, 0)),
        ],
        out_specs=[
            pl.BlockSpec((rows_per_step, vocab), lambda i: (i, 0)),
            pl.BlockSpec((rows_per_step, 128), lambda i: (i, 0)),
        ],
        out_shape=[
            jax.ShapeDtypeStruct((rows, vocab), jnp.float32),
            jax.ShapeDtypeStruct((rows, 128), jnp.int32),
        ],
        interpret=interpret,
    )(x2)

    output_ids = ids2[:, 0].reshape(b0, b1)
    y_soft = y2.reshape(b0, b1, vocab)
    eff_temperature = jnp.asarray([1.0], dtype=x.dtype)
    return output_ids, y_soft, eff_temperature


# speedup vs baseline: 1.1973x; 1.1973x over previous
import functools

import jax
import jax.numpy as jnp
from jax import lax
from jax.experimental import pallas as pl

_KU = (0x6D3E048F, 0x1022172D)
_KS = (0x03D7B32D, 0xADD083F4)

_EPS = 1e-12
_TINY = float(jnp.finfo(jnp.float32).tiny)
_ROT = ((13, 15, 26, 6), (17, 29, 16, 24))

_ROWS_PER_STEP = 8
_CHUNK = 1024


def _threefry_bits(cnt, k0, k1):
    ks = (k0, k1, k0 ^ k1 ^ 0x1BD11BDA)
    x0 = jnp.full(cnt.shape, jnp.uint32(k0))
    x1 = cnt + jnp.uint32(k1)
    for b in range(5):
        for r in _ROT[b % 2]:
            x0 = x0 + x1
            x1 = (x1 << jnp.uint32(r)) | (x1 >> jnp.uint32(32 - r))
            x1 = x1 ^ x0
        x0 = x0 + jnp.uint32(ks[(b + 1) % 3])
        x1 = x1 + jnp.uint32((ks[(b + 2) % 3] + b + 1) & 0xFFFFFFFF)
    return x0 ^ x1


def _bits_to_unit_float(bits):
    f = lax.bitcast_convert_type(
        (bits >> jnp.uint32(9)) | jnp.uint32(0x3F800000), jnp.float32
    )
    return f - jnp.float32(1.0)


def _gumbel_logits(x, cnt):
    f = _bits_to_unit_float(_threefry_bits(cnt, *_KU))
    u = f * jnp.float32(0.999 - _EPS) + jnp.float32(_EPS)
    return x + (-jnp.log(-jnp.log(u)))


def _gumbel2(cnt):
    f = _bits_to_unit_float(_threefry_bits(cnt, *_KS))
    u = jnp.maximum(
        f * jnp.float32(1.0 - _TINY) + jnp.float32(_TINY), jnp.float32(_TINY)
    )
    return -jnp.log(-jnp.log(u))


def _stgs_body(x_ref, y_ref, ids_ref, *, vocab):
    i = pl.program_id(0)
    R, C = _ROWS_PER_STEP, _CHUNK
    n_chunks = vocab // C
    tail_start = n_chunks * C
    tail = vocab - tail_start

    row_base = (
        lax.broadcasted_iota(jnp.int32, (R, 1), 0) + i * R
    ) * vocab

    def cnt_for(col):
        return (row_base + col).astype(jnp.uint32)

    def col_iota(j, width):
        return lax.broadcasted_iota(jnp.int32, (R, width), 1) + j * C

    def chunk_a(j, carry, width):
        m, s = carry
        gl = _gumbel_logits(
            x_ref[:, j * C:j * C + width], cnt_for(col_iota(j, width))
        )
        y_ref[:, j * C:j * C + width] = gl
        m_new = jnp.maximum(m, jnp.max(gl, axis=1, keepdims=True))
        s = s * jnp.exp(m - m_new) + jnp.sum(
            jnp.exp(gl - m_new), axis=1, keepdims=True
        )
        return m_new, s

    carry = (
        jnp.full((R, 1), -jnp.inf, jnp.float32),
        jnp.zeros((R, 1), jnp.float32),
    )
    for j in range(n_chunks):
        carry = chunk_a(j, carry, C)
    if tail:
        carry = chunk_a(n_chunks, carry, tail)
    m, s = carry
    rinv = jnp.float32(1.0) / s

    def chunk_vals(j, width):
        y = jnp.exp(y_ref[:, j * C:j * C + width] - m) * rinv
        y_ref[:, j * C:j * C + width] = y
        col = col_iota(j, width)
        vals = jnp.log(y + jnp.float32(_EPS)) + _gumbel2(cnt_for(col))
        return vals, col

    vals_run = jnp.full((R, C), -jnp.inf, jnp.float32)
    col_run = jnp.zeros((R, C), jnp.int32)
    for j in range(n_chunks):
        vals, col = chunk_vals(j, C)
        col_run = jnp.where(vals > vals_run, col, col_run)
        vals_run = jnp.maximum(vals_run, vals)

    vmax = jnp.max(vals_run, axis=1, keepdims=True)
    vidx = jnp.min(
        jnp.where(vals_run == vmax, col_run, jnp.int32(vocab)),
        axis=1,
        keepdims=True,
    )
    if tail:
        vals, col = chunk_vals(n_chunks, tail)
        lm = jnp.max(vals, axis=1, keepdims=True)
        lidx = jnp.min(
            jnp.where(vals == lm, col, jnp.int32(vocab)), axis=1, keepdims=True
        )
        upd = lm > vmax
        vidx = jnp.where(upd, lidx, vidx)

    ids_ref[...] = jnp.broadcast_to(vidx, (R, 128))


@functools.partial(jax.jit, static_argnames=("interpret",))
def kernel(x, interpret=False):
    b0, b1, vocab = x.shape
    rows = b0 * b1
    rows_per_step = _ROWS_PER_STEP
    grid = rows // rows_per_step
    x2 = x.reshape(rows, vocab)

    y2, ids2 = pl.pallas_call(
        functools.partial(_stgs_body, vocab=vocab),
        grid=(grid,),
        in_specs=[
            pl.BlockSpec((rows_per_step, vocab), lambda i: (i, 0)),
        ],
        out_specs=[
            pl.BlockSpec((rows_per_step, vocab), lambda i: (i, 0)),
            pl.BlockSpec((rows_per_step, 128), lambda i: (i, 0)),
        ],
        out_shape=[
            jax.ShapeDtypeStruct((rows, vocab), jnp.float32),
            jax.ShapeDtypeStruct((rows, 128), jnp.int32),
        ],
        interpret=interpret,
    )(x2)

    output_ids = ids2[:, 0].reshape(b0, b1)
    y_soft = y2.reshape(b0, b1, vocab)
    eff_temperature = jnp.asarray([1.0], dtype=x.dtype)
    return output_ids, y_soft, eff_temperature
